# baseline (speedup 1.0000x reference)
import jax
import jax.numpy as jnp
from jax import lax
from jax.experimental import pallas as pl
from jax.experimental.pallas import tpu as pltpu
from jax.experimental.pallas import tpu_sc as plsc

_B, _S, _D = 4, 4096, 1024
_NW = 32
_ELEMS_W = (_B * _S * _D) // _NW
_CHE = 8 * _D
_NCHUNK = _ELEMS_W // _CHE
_NSLOT = 6
_LEAD = 4


def _sc_body(x_hbm, pe_hbm, out_hbm, *refs):
    xbs, pbs = refs[0:_NSLOT], refs[_NSLOT:2 * _NSLOT]
    lsems, ssems = refs[2 * _NSLOT:3 * _NSLOT], refs[3 * _NSLOT:4 * _NSLOT]

    c = lax.axis_index("c")
    s = lax.axis_index("s")
    wid = s * 2 + c
    base = wid * _ELEMS_W
    pe_base = lax.rem(wid, 8) * _ELEMS_W

    loads, stores = {}, {}

    def start_load(i):
        sl = i % _NSLOT
        cx = pltpu.make_async_copy(
            x_hbm.at[pl.ds(base + i * _CHE, _CHE)], xbs[sl], lsems[sl])
        cp = pltpu.make_async_copy(
            pe_hbm.at[pl.ds(pe_base + i * _CHE, _CHE)], pbs[sl], lsems[sl])
        cx.start()
        cp.start()
        loads[i] = (cx, cp)

    def start_store(i):
        sl = i % _NSLOT
        cs = pltpu.make_async_copy(
            xbs[sl], out_hbm.at[pl.ds(base + i * _CHE, _CHE)], ssems[sl])
        cs.start()
        stores[i] = cs

    for i in range(_LEAD):
        start_load(i)
    for i in range(_NCHUNK):
        nxt = i + _LEAD
        if nxt < _NCHUNK:
            prev = nxt - _NSLOT
            if prev >= 0:
                stores.pop(prev).wait()
            start_load(nxt)
        cx, cp = loads.pop(i)
        cx.wait()
        cp.wait()
        sl = i % _NSLOT

        @plsc.parallel_loop(0, _CHE, step=16, unroll=8)
        def _add(j, xb=xbs[sl], pb=pbs[sl]):
            xb[pl.ds(j, 16)] = xb[pl.ds(j, 16)] + pb[pl.ds(j, 16)]

        start_store(i)
    for i in sorted(stores):
        stores[i].wait()
    stores.clear()


def kernel(x, pe_table):
    xf = x.reshape(_B * _S * _D)
    pef = pe_table.reshape(-1)
    mesh = plsc.VectorSubcoreMesh(core_axis_name="c", subcore_axis_name="s")
    out = pl.kernel(
        _sc_body,
        out_type=jax.ShapeDtypeStruct((_B * _S * _D,), jnp.float32),
        mesh=mesh,
        scratch_types=(
            [pltpu.VMEM((_CHE,), jnp.float32) for _ in range(2 * _NSLOT)]
            + [pltpu.SemaphoreType.DMA for _ in range(2 * _NSLOT)]
        ),
    )(xf, pef)
    return out.reshape(_B, _S, _D)


# speedup vs baseline: 2.5678x; 2.5678x over previous
import jax
import jax.numpy as jnp
from jax import lax
from jax.experimental import pallas as pl
from jax.experimental.pallas import tpu as pltpu
from jax.experimental.pallas import tpu_sc as plsc

_B, _S, _D = 4, 4096, 1024
_NW = 32
_ROWS_W = (_B * _S) // _NW
_WPB = _S // _ROWS_W
_CH = 16
_NCHUNK = _ROWS_W // _CH
_NSLOT = 3
_LEAD = 2


def _sc_body(x_hbm, pe_hbm, out_hbm, *refs):
    xbs, pbs = refs[0:_NSLOT], refs[_NSLOT:2 * _NSLOT]
    lsems, ssems = refs[2 * _NSLOT:3 * _NSLOT], refs[3 * _NSLOT:4 * _NSLOT]

    c = lax.axis_index("c")
    s = lax.axis_index("s")
    wid = s * 2 + c
    b = wid // _WPB
    s0 = lax.rem(wid, _WPB) * _ROWS_W

    loads, stores = {}, {}

    def start_load(i):
        sl = i % _NSLOT
        r0 = s0 + i * _CH
        cx = pltpu.make_async_copy(
            x_hbm.at[b, pl.ds(r0, _CH)], xbs[sl], lsems[sl])
        cp = pltpu.make_async_copy(
            pe_hbm.at[pl.ds(r0, _CH)], pbs[sl], lsems[sl])
        cx.start()
        cp.start()
        loads[i] = (cx, cp)

    def start_store(i):
        sl = i % _NSLOT
        cs = pltpu.make_async_copy(
            xbs[sl], out_hbm.at[b, pl.ds(s0 + i * _CH, _CH)], ssems[sl])
        cs.start()
        stores[i] = cs

    for i in range(_LEAD):
        start_load(i)
    for i in range(_NCHUNK):
        nxt = i + _LEAD
        if nxt < _NCHUNK:
            prev = nxt - _NSLOT
            if prev >= 0:
                stores.pop(prev).wait()
            start_load(nxt)
        cx, cp = loads.pop(i)
        cx.wait()
        cp.wait()
        sl = i % _NSLOT

        @plsc.parallel_loop(0, _CH * _D, step=16, unroll=8)
        def _add(j, xb=xbs[sl], pb=pbs[sl]):
            r = j >> 10
            col = pl.multiple_of(j & (_D - 1), 16)
            xb[r, pl.ds(col, 16)] = xb[r, pl.ds(col, 16)] + pb[r, pl.ds(col, 16)]

        start_store(i)
    for i in sorted(stores):
        stores[i].wait()
    stores.clear()


def kernel(x, pe_table):
    mesh = plsc.VectorSubcoreMesh(core_axis_name="c", subcore_axis_name="s")
    return pl.kernel(
        _sc_body,
        out_type=jax.ShapeDtypeStruct((_B, _S, _D), jnp.float32),
        mesh=mesh,
        scratch_types=(
            [pltpu.VMEM((_CH, _D), jnp.float32) for _ in range(2 * _NSLOT)]
            + [pltpu.SemaphoreType.DMA for _ in range(2 * _NSLOT)]
        ),
    )(x, pe_table)


# speedup vs baseline: 2.9332x; 1.1423x over previous
import jax
import jax.numpy as jnp
from jax import lax
from jax.experimental import pallas as pl
from jax.experimental.pallas import tpu as pltpu
from jax.experimental.pallas import tpu_sc as plsc

_B, _S, _D = 4, 4096, 1024
_NW = 32
_SROWS_W = _S // _NW
_CH = 16
_NSCH = _SROWS_W // _CH
_NT = _NSCH * _B
_NSLOT = 3
_NPSLOT = 2
_LEAD = 2


def _sc_body(x_hbm, pe_hbm, out_hbm, *refs):
    xbs = refs[0:_NSLOT]
    pbs = refs[_NSLOT:_NSLOT + _NPSLOT]
    lsems = refs[_NSLOT + _NPSLOT:2 * _NSLOT + _NPSLOT]
    psems = refs[2 * _NSLOT + _NPSLOT:2 * _NSLOT + 2 * _NPSLOT]
    ssems = refs[2 * _NSLOT + 2 * _NPSLOT:3 * _NSLOT + 2 * _NPSLOT]

    c = lax.axis_index("c")
    s = lax.axis_index("s")
    wid = s * 2 + c
    s0 = wid * _SROWS_W

    xloads, peloads, stores = {}, {}, {}

    def start_xload(t):
        i, b = divmod(t, _B)
        sl = t % _NSLOT
        cx = pltpu.make_async_copy(
            x_hbm.at[b, pl.ds(s0 + i * _CH, _CH)], xbs[sl], lsems[sl])
        cx.start()
        xloads[t] = cx

    def start_peload(i):
        sl = i % _NPSLOT
        cp = pltpu.make_async_copy(
            pe_hbm.at[pl.ds(s0 + i * _CH, _CH)], pbs[sl], psems[sl])
        cp.start()
        peloads[i] = cp

    def start_store(t):
        i, b = divmod(t, _B)
        sl = t % _NSLOT
        cs = pltpu.make_async_copy(
            xbs[sl], out_hbm.at[b, pl.ds(s0 + i * _CH, _CH)], ssems[sl])
        cs.start()
        stores[t] = cs

    for t in range(_LEAD):
        start_xload(t)
    start_peload(0)
    start_peload(1)
    for t in range(_NT):
        i, b = divmod(t, _B)
        nxt = t + _LEAD
        if nxt < _NT:
            prev = nxt - _NSLOT
            if prev >= 0:
                stores.pop(prev).wait()
            start_xload(nxt)
        if b == 0:
            peloads.pop(i).wait()
        xloads.pop(t).wait()
        sl = t % _NSLOT
        pi = i % _NPSLOT

        @plsc.parallel_loop(0, _CH * _D, step=16, unroll=8)
        def _add(j, xb=xbs[sl], pb=pbs[pi]):
            r = j >> 10
            col = pl.multiple_of(j & (_D - 1), 16)
            xb[r, pl.ds(col, 16)] = xb[r, pl.ds(col, 16)] + pb[r, pl.ds(col, 16)]

        start_store(t)
        if b == _B - 1 and i + _NPSLOT < _NSCH:
            start_peload(i + _NPSLOT)
    for t in sorted(stores):
        stores[t].wait()
    stores.clear()


def kernel(x, pe_table):
    mesh = plsc.VectorSubcoreMesh(core_axis_name="c", subcore_axis_name="s")
    return pl.kernel(
        _sc_body,
        out_type=jax.ShapeDtypeStruct((_B, _S, _D), jnp.float32),
        mesh=mesh,
        scratch_types=(
            [pltpu.VMEM((_CH, _D), jnp.float32) for _ in range(_NSLOT)]
            + [pltpu.VMEM((_CH, _D), jnp.float32) for _ in range(_NPSLOT)]
            + [pltpu.SemaphoreType.DMA for _ in range(2 * _NSLOT + _NPSLOT)]
        ),
    )(x, pe_table)


# speedup vs baseline: 2.9444x; 1.0038x over previous
import jax
import jax.numpy as jnp
from jax import lax
from jax.experimental import pallas as pl
from jax.experimental.pallas import tpu as pltpu
from jax.experimental.pallas import tpu_sc as plsc

_B, _S, _D = 4, 4096, 1024
_NW = 32
_SROWS_W = _S // _NW
_CH = 16
_NSCH = _SROWS_W // _CH
_NT = _NSCH * _B
_NSLOT = 4
_NPSLOT = 2
_LEAD = 3


def _sc_body(x_hbm, pe_hbm, out_hbm, *refs):
    xbs = refs[0:_NSLOT]
    pbs = refs[_NSLOT:_NSLOT + _NPSLOT]
    lsems = refs[_NSLOT + _NPSLOT:2 * _NSLOT + _NPSLOT]
    psems = refs[2 * _NSLOT + _NPSLOT:2 * _NSLOT + 2 * _NPSLOT]
    ssems = refs[2 * _NSLOT + 2 * _NPSLOT:3 * _NSLOT + 2 * _NPSLOT]

    c = lax.axis_index("c")
    s = lax.axis_index("s")
    wid = s * 2 + c
    s0 = wid * _SROWS_W

    xloads, peloads, stores = {}, {}, {}

    def start_xload(t):
        i, b = divmod(t, _B)
        sl = t % _NSLOT
        cx = pltpu.make_async_copy(
            x_hbm.at[b, pl.ds(s0 + i * _CH, _CH)], xbs[sl], lsems[sl])
        cx.start()
        xloads[t] = cx

    def start_peload(i):
        sl = i % _NPSLOT
        cp = pltpu.make_async_copy(
            pe_hbm.at[pl.ds(s0 + i * _CH, _CH)], pbs[sl], psems[sl])
        cp.start()
        peloads[i] = cp

    def start_store(t):
        i, b = divmod(t, _B)
        sl = t % _NSLOT
        cs = pltpu.make_async_copy(
            xbs[sl], out_hbm.at[b, pl.ds(s0 + i * _CH, _CH)], ssems[sl])
        cs.start()
        stores[t] = cs

    for t in range(_LEAD):
        start_xload(t)
    start_peload(0)
    start_peload(1)
    for t in range(_NT):
        i, b = divmod(t, _B)
        nxt = t + _LEAD
        if nxt < _NT:
            prev = nxt - _NSLOT
            if prev >= 0:
                stores.pop(prev).wait()
            start_xload(nxt)
        if b == 0:
            peloads.pop(i).wait()
        xloads.pop(t).wait()
        sl = t % _NSLOT
        pi = i % _NPSLOT

        @plsc.parallel_loop(0, _CH * _D, step=16, unroll=8)
        def _add(j, xb=xbs[sl], pb=pbs[pi]):
            r = j >> 10
            col = pl.multiple_of(j & (_D - 1), 16)
            xb[r, pl.ds(col, 16)] = xb[r, pl.ds(col, 16)] + pb[r, pl.ds(col, 16)]

        start_store(t)
        if b == _B - 1 and i + _NPSLOT < _NSCH:
            start_peload(i + _NPSLOT)
    for t in sorted(stores):
        stores[t].wait()
    stores.clear()


def kernel(x, pe_table):
    mesh = plsc.VectorSubcoreMesh(core_axis_name="c", subcore_axis_name="s")
    return pl.kernel(
        _sc_body,
        out_type=jax.ShapeDtypeStruct((_B, _S, _D), jnp.float32),
        mesh=mesh,
        scratch_types=(
            [pltpu.VMEM((_CH, _D), jnp.float32) for _ in range(_NSLOT)]
            + [pltpu.VMEM((_CH, _D), jnp.float32) for _ in range(_NPSLOT)]
            + [pltpu.SemaphoreType.DMA for _ in range(2 * _NSLOT + _NPSLOT)]
        ),
    )(x, pe_table)
